# baseline (speedup 1.0000x reference)
import functools

import jax
import jax.numpy as jnp
from jax import lax
from jax.experimental import pallas as pl
from jax.experimental.pallas import tpu as pltpu
from jax.experimental.pallas import tpu_sc as plsc

_N, _IN_DIM, _HID, _K, _TOP_M = 16384, 512, 256, 1024, 512
_TN = 1024


def _norm_body(em_ref, et_ref, ep_ref, om_ref, ot_ref, op_ref):
    for r, o in ((em_ref, om_ref), (et_ref, ot_ref), (ep_ref, op_ref)):
        e = r[...]
        n = jnp.sqrt(jnp.sum(e * e, axis=-1, keepdims=True))
        o[...] = e / (n + 1e-8)


def _normalize_codebooks(Em, Et, Ep):
    out = jax.ShapeDtypeStruct((_K, _HID), jnp.float32)
    return pl.pallas_call(
        _norm_body,
        out_shape=(out, out, out),
    )(Em, Et, Ep)


def _main_body(h0_ref, W1_ref, b1_ref, W2_ref, b2_ref,
               Wm_ref, bm_ref, Wt_ref, bt_ref, Wp_ref, bp_ref,
               Enm_ref, Ent_ref, Enp_ref, Wk_ref, bk_ref,
               km_ref, kt_ref, kp_ref, alpha_ref):
    H = jax.nn.relu(jnp.dot(h0_ref[...], W1_ref[...]) + b1_ref[...])
    H = jax.nn.relu(jnp.dot(H, W2_ref[...]) + b2_ref[...])
    for W_ref, b_ref, En_ref, out_ref in (
            (Wm_ref, bm_ref, Enm_ref, km_ref),
            (Wt_ref, bt_ref, Ent_ref, kt_ref),
            (Wp_ref, bp_ref, Enp_ref, kp_ref)):
        z = jnp.dot(H, W_ref[...]) + b_ref[...]
        nrm = jnp.sqrt(jnp.sum(z * z, axis=-1, keepdims=True))
        zn = z / (nrm + 1e-8)
        dist = jax.lax.dot_general(
            zn, En_ref[...], (((1,), (1,)), ((), ())))
        m = jnp.min(dist, axis=1, keepdims=True)
        iota = jax.lax.broadcasted_iota(jnp.int32, dist.shape, 1)
        idx = jnp.min(jnp.where(dist == m, iota, _K), axis=1, keepdims=True)
        out_ref[...] = idx
    xk = jnp.dot(H, Wk_ref[...]) + bk_ref[...]
    alpha_ref[...] = jax.nn.sigmoid(xk)


def _fused_dense(h0, W1, b1, W2, b2, Wm, bm, Wt, bt, Wp, bp,
                 Enm, Ent, Enp, Wk, bk):
    grid = (_N // _TN,)
    row = lambda i: (i, 0)
    rep = lambda i: (0, 0)
    col_i32 = jax.ShapeDtypeStruct((_N, 1), jnp.int32)
    col_f32 = jax.ShapeDtypeStruct((_N, 1), jnp.float32)
    in_specs = [
        pl.BlockSpec((_TN, _IN_DIM), row),
        pl.BlockSpec((_IN_DIM, _HID), rep),
        pl.BlockSpec((1, _HID), rep),
        pl.BlockSpec((_HID, _HID), rep),
        pl.BlockSpec((1, _HID), rep),
    ]
    for _ in range(3):
        in_specs += [pl.BlockSpec((_HID, _HID), rep),
                     pl.BlockSpec((1, _HID), rep)]
    in_specs += [pl.BlockSpec((_K, _HID), rep)] * 3
    in_specs += [pl.BlockSpec((_HID, 1), rep),
                 pl.BlockSpec((1, 1), rep)]
    out_specs = [pl.BlockSpec((_TN, 1), row)] * 4
    km, kt, kp, alpha = pl.pallas_call(
        _main_body,
        grid=grid,
        in_specs=in_specs,
        out_specs=out_specs,
        out_shape=(col_i32, col_i32, col_i32, col_f32),
    )(h0, W1, b1.reshape(1, _HID), W2, b2.reshape(1, _HID),
      Wm, bm.reshape(1, _HID), Wt, bt.reshape(1, _HID),
      Wp, bp.reshape(1, _HID), Enm, Ent, Enp,
      Wk, bk.reshape(1, 1))
    return km, kt, kp, alpha


_NT = 16
_SHARD = _N // _NT
_NV = _SHARD // 16
_OUTP = _TOP_M // _NT
_MESH = None


def _li():
    return lax.broadcasted_iota(jnp.int32, (16,), 0)


def _f16(x):
    return jnp.full((16,), x, jnp.int32)


def _scal(x):
    return jnp.max(x) if getattr(x, "ndim", 0) else x


def _topk_body(alpha_hbm, km_hbm, kt_hbm, kp_hbm,
               keyidx_hbm, sm_hbm, st_hbm, sp_hbm,
               keys_v, hist_v, tot_v, gtot_v, histall_v,
               selk_v, seli_v, eqi_v, myidx_v, zeros_v,
               cnt_v, allcnt_v, allk_v, alli_v, idx32_v, g_v,
               hist_s, cnt_s, selk_s, seli_s, fin_s):
    s = lax.axis_index("s")
    c = lax.axis_index("c")
    li = _li()
    ones = jnp.ones((16,), jnp.int32)
    zero = jnp.zeros((16,), jnp.int32)

    pltpu.sync_copy(alpha_hbm.at[pl.ds(s * _SHARD, _SHARD)], keys_v)

    def _z_hist(j, _):
        plsc.store_scatter(hist_v, [16 * j + li], zero)
        return 0
    lax.fori_loop(0, 4096 // 16, _z_hist, 0)

    def _z512(ref):
        def _b(j, _):
            plsc.store_scatter(ref, [16 * j + li], zero)
            return 0
        lax.fori_loop(0, _TOP_M // 16, _b, 0)
    _z512(zeros_v)

    need = jnp.int32(_TOP_M)
    prefix = jnp.int32(0)
    for r, shift in enumerate((24, 16, 8, 0)):
        def _hist(j, carry):
            k = plsc.bitcast(plsc.load_gather(keys_v, [16 * j + li]),
                             jnp.int32)
            d = jnp.bitwise_and(
                lax.shift_right_logical(k, jnp.full((16,), shift, jnp.int32)),
                255)
            hidx = li * 256 + d
            if r == 0:
                plsc.addupdate_scatter(hist_v, [hidx], ones)
            else:
                act = lax.shift_right_logical(
                    k, jnp.full((16,), shift + 8, jnp.int32)) == carry
                plsc.addupdate_scatter(hist_v, [hidx], ones, mask=act)
            return carry
        phi = lax.shift_right_logical(prefix, jnp.int32(min(shift + 8, 31)))
        lax.fori_loop(0, _NV, _hist, phi)
        def _comb(j, _):
            acc = zero
            for l in range(16):
                acc = acc + plsc.load_gather(hist_v, [l * 256 + 16 * j + li])
            plsc.store_scatter(tot_v, [16 * j + li], acc)
            return 0
        lax.fori_loop(0, 16, _comb, 0)
        pltpu.sync_copy(tot_v, hist_s.at[r, s])
        plsc.subcore_barrier()
        pltpu.sync_copy(hist_s.at[r], histall_v)
        def _comb2(j, _):
            acc = zero
            for t in range(_NT):
                acc = acc + plsc.load_gather(
                    histall_v, [_f16(t), 16 * j + li])
            plsc.store_scatter(gtot_v, [16 * j + li], acc)
            return 0
        lax.fori_loop(0, 16, _comb2, 0)
        tvecs = [gtot_v[pl.ds(16 * j, 16)] for j in range(16)]
        gsums = [jnp.sum(t) for t in tvecs]
        D = jnp.int32(0)
        above = jnp.int32(0)
        acc_hi = jnp.int32(0)
        for j in range(15, -1, -1):
            t_j = tvecs[j]
            g_j = gsums[j]
            crossing = jnp.logical_and(acc_hi < need, acc_hi + g_j >= need)
            rt = lax.rev(t_j, (0,))
            cs = plsc.cumsum(rt)
            nb = need - acc_hi
            i0 = _scal(plsc.all_reduce_ffs(cs >= nb))
            e1 = jnp.sum(jnp.where(li == i0, cs, 0))
            e2 = jnp.sum(jnp.where(li == i0, rt, 0))
            D = jnp.where(crossing, 16 * j + 15 - i0, D)
            above = jnp.where(crossing, acc_hi + e1 - e2, above)
            acc_hi = acc_hi + g_j
        need = need - above
        prefix = jnp.bitwise_or(
            prefix, lax.shift_left(D, jnp.int32(shift)))
        if r < 3:
            def _rz(j, _):
                plsc.store_scatter(hist_v, [16 * j + li], zero)
                return 0
            lax.fori_loop(0, 4096 // 16, _rz, 0)

    T = prefix
    eneed = need
    G = jnp.int32(_TOP_M) - eneed

    @pl.when(s == 0)
    def _():
        pltpu.sync_copy(zeros_v, selk_s.at[pl.ds(0, _TOP_M)])

    def _sel(j, carry):
        off_gt, off_eq = carry
        k = plsc.bitcast(plsc.load_gather(keys_v, [16 * j + li]), jnp.int32)
        gidx = s * _SHARD + 16 * j + li
        mgt = k > T
        meq = k == T
        mgt_i = mgt.astype(jnp.int32)
        meq_i = meq.astype(jnp.int32)
        ex_gt = plsc.cumsum(mgt_i) - mgt_i
        plsc.store_scatter(selk_v, [off_gt + ex_gt], k, mask=mgt)
        plsc.store_scatter(seli_v, [off_gt + ex_gt], gidx, mask=mgt)
        ex_eq = plsc.cumsum(meq_i) - meq_i
        meq_c = jnp.logical_and(meq, off_eq + ex_eq < _TOP_M)
        plsc.store_scatter(eqi_v, [off_eq + ex_eq], gidx, mask=meq_c)
        return (off_gt + jnp.sum(mgt_i), off_eq + jnp.sum(meq_i))
    cnt_gt, cnt_eq = lax.fori_loop(0, _NV, _sel, (jnp.int32(0), jnp.int32(0)))

    cnt_v[...] = jnp.where(li == 0, cnt_gt, jnp.where(li == 1, cnt_eq, 0))
    pltpu.sync_copy(cnt_v, cnt_s.at[s])
    plsc.subcore_barrier()
    pltpu.sync_copy(cnt_s, allcnt_v)
    gt_base = jnp.int32(0)
    eq_base = jnp.int32(0)
    for t in range(_NT):
        row = allcnt_v[t]
        gt_t = row[0]
        eq_t = row[1]
        gt_base = gt_base + jnp.where(t < s, gt_t, 0)
        eq_base = eq_base + jnp.where(t < s, eq_t, 0)

    def _pos(j, _):
        slot = 16 * j + li
        p = jnp.where(slot < cnt_gt, gt_base + slot, _TOP_M + s)
        plsc.store_scatter(myidx_v, [slot], p)
        return 0
    lax.fori_loop(0, _TOP_M // 16, _pos, 0)
    pltpu.sync_copy(selk_v, selk_s.at[myidx_v])
    pltpu.sync_copy(seli_v, seli_s.at[myidx_v])
    plsc.subcore_barrier()
    pltpu.sync_copy(selk_s.at[pl.ds(0, _TOP_M)], allk_v)
    pltpu.sync_copy(seli_s.at[pl.ds(0, _TOP_M)], alli_v)

    nvg = lax.div(G + 15, jnp.int32(16))

    def _rank(e, _):
        ek = plsc.load_gather(selk_v, [_f16(e)])
        ei = plsc.load_gather(seli_v, [_f16(e)])

        def _cmp(j, acc):
            bk = plsc.load_gather(allk_v, [16 * j + li])
            bi = plsc.load_gather(alli_v, [16 * j + li])
            hit = jnp.logical_or(
                bk > ek, jnp.logical_and(bk == ek, bi < ei))
            return acc + hit.astype(jnp.int32)
        accv = lax.fori_loop(0, nvg, _cmp, zero)
        rank = jnp.sum(accv)
        plsc.store_scatter(myidx_v, [_f16(e)], _f16(rank), mask=li == 0)
        return 0
    def _pad(j, _):
        plsc.store_scatter(myidx_v, [16 * j + li], _f16(_TOP_M + s))
        return 0
    lax.fori_loop(0, _TOP_M // 16, _pad, 0)
    lax.fori_loop(0, cnt_gt, _rank, 0)

    k_eq = jnp.clip(eneed - eq_base, 0, cnt_eq)

    def _eq(j, _):
        slot = 16 * j + li
        m = slot < k_eq
        v = plsc.load_gather(eqi_v, [slot])
        plsc.store_scatter(seli_v, [cnt_gt + slot], v, mask=m)
        plsc.store_scatter(myidx_v, [cnt_gt + slot],
                           G + eq_base + slot, mask=m)
        return 0
    lax.fori_loop(0, _TOP_M // 16, _eq, 0)

    pltpu.sync_copy(seli_v, fin_s.at[myidx_v])
    plsc.subcore_barrier()

    @pl.when(c == 0)
    def _():
        pltpu.sync_copy(fin_s.at[pl.ds(s * _OUTP, _OUTP)], idx32_v)
        pltpu.sync_copy(idx32_v, keyidx_hbm.at[pl.ds(s * _OUTP, _OUTP)])
        for src, dst in ((km_hbm, sm_hbm), (kt_hbm, st_hbm),
                         (kp_hbm, sp_hbm)):
            pltpu.sync_copy(src.at[idx32_v], g_v)
            pltpu.sync_copy(g_v, dst.at[pl.ds(s * _OUTP, _OUTP)])


def _topk_gather(alpha, km, kt, kp):
    mesh = plsc.VectorSubcoreMesh(core_axis_name="c", subcore_axis_name="s")
    o = jax.ShapeDtypeStruct((_TOP_M,), jnp.int32)
    f = pl.kernel(
        _topk_body,
        mesh=mesh,
        out_type=(o, o, o, o),
        compiler_params=pltpu.CompilerParams(needs_layout_passes=False),
        scratch_types=[
            pltpu.VMEM((_SHARD,), jnp.float32),
            pltpu.VMEM((4096,), jnp.int32),
            pltpu.VMEM((256,), jnp.int32),
            pltpu.VMEM((256,), jnp.int32),
            pltpu.VMEM((_NT, 256), jnp.int32),
            pltpu.VMEM((_TOP_M,), jnp.int32),
            pltpu.VMEM((_TOP_M,), jnp.int32),
            pltpu.VMEM((_TOP_M,), jnp.int32),
            pltpu.VMEM((_TOP_M,), jnp.int32),
            pltpu.VMEM((_TOP_M,), jnp.int32),
            pltpu.VMEM((16,), jnp.int32),
            pltpu.VMEM((_NT, 16), jnp.int32),
            pltpu.VMEM((_TOP_M,), jnp.int32),
            pltpu.VMEM((_TOP_M,), jnp.int32),
            pltpu.VMEM((_OUTP,), jnp.int32),
            pltpu.VMEM((_OUTP,), jnp.int32),
            pltpu.VMEM_SHARED((4, _NT, 256), jnp.int32),
            pltpu.VMEM_SHARED((_NT, 16), jnp.int32),
            pltpu.VMEM_SHARED((_TOP_M + _NT,), jnp.int32),
            pltpu.VMEM_SHARED((_TOP_M + _NT,), jnp.int32),
            pltpu.VMEM_SHARED((_TOP_M + _NT,), jnp.int32),
        ],
    )
    return f(alpha, km, kt, kp)


def kernel(h0, W1, b1, W2, b2, Wm, bm, Wt, bt, Wp, bp, Em, Et, Ep, Wk, bk):
    Enm, Ent, Enp = _normalize_codebooks(Em, Et, Ep)
    km, kt, kp, alpha = _fused_dense(
        h0, W1, b1, W2, b2, Wm, bm, Wt, bt, Wp, bp, Enm, Ent, Enp, Wk, bk)
    codes = jnp.concatenate([km, kt, kp], axis=1)
    key_idx, sm, st, sp = _topk_gather(
        alpha.reshape(_N), km.reshape(_N), kt.reshape(_N), kp.reshape(_N))
    sel = jnp.stack([sm, st, sp], axis=1)
    return (codes, key_idx, sel)
